# baseline (speedup 1.0000x reference)
import functools

import jax
import jax.numpy as jnp
from jax import lax
from jax.experimental import pallas as pl
from jax.experimental.pallas import tpu as pltpu
from jax.experimental.pallas import tpu_sc as plsc

NUM_CORES = 2
NUM_SUBCORES = 16
NW = NUM_CORES * NUM_SUBCORES
CHUNK = 128
KB = 5


@functools.partial(jax.jit, static_argnums=(3, 4))
def _segsum(h, src, dst, n_pad, cpt):
  d = h.shape[1]
  rows_per_tile = n_pad // NUM_SUBCORES
  assert rows_per_tile == KB * CHUNK

  mesh = plsc.VectorSubcoreMesh(
      core_axis_name="c", subcore_axis_name="s",
      num_cores=NUM_CORES, num_subcores=NUM_SUBCORES)

  @functools.partial(
      pl.kernel,
      out_type=jax.ShapeDtypeStruct((NUM_CORES, n_pad, d), jnp.float32),
      mesh=mesh,
      scratch_types=[
          pltpu.VMEM((cpt, CHUNK), jnp.int32),
          pltpu.VMEM((cpt, CHUNK), jnp.int32),
          pltpu.VMEM((CHUNK, d), jnp.float32),
          pltpu.VMEM_SHARED((n_pad, d), jnp.float32),
          pltpu.SemaphoreType.DMA,
      ],
  )
  def seg(h_hbm, src_hbm, dst_hbm, out_hbm, src_v, dst_v, msgs, agg, sem):
    cid = lax.axis_index("c")
    sid = lax.axis_index("s")
    wid = cid * NUM_SUBCORES + sid

    pltpu.sync_copy(src_hbm.at[wid], src_v)
    pltpu.sync_copy(dst_hbm.at[wid], dst_v)

    zero = jnp.zeros((16,), jnp.float32)

    def zbody(r, carry):
      for k in range(d // 16):
        msgs[r, pl.ds(k * 16, 16)] = zero
      return carry

    lax.fori_loop(0, CHUNK, zbody, 0)
    base = sid * rows_per_tile

    def zcopy(i, carry):
      pltpu.sync_copy(msgs, agg.at[pl.ds(base + i * CHUNK, CHUNK)])
      return carry

    lax.fori_loop(0, KB, zcopy, 0)
    plsc.subcore_barrier()

    def body(j, carry):
      pltpu.async_copy(h_hbm.at[src_v.at[j]], msgs, sem).wait()
      pltpu.sync_copy(msgs, agg.at[dst_v.at[j]], add=True)
      return carry

    lax.fori_loop(0, cpt, body, 0)
    plsc.subcore_barrier()

    def ocopy(i, carry):
      sl = pl.ds(base + i * CHUNK, CHUNK)
      pltpu.sync_copy(agg.at[sl], out_hbm.at[cid, sl])
      return carry

    lax.fori_loop(0, KB, ocopy, 0)

  return seg(h, src, dst)


@functools.partial(jax.jit, static_argnums=(3,))
def _linear(p, w, b, leaky):
  n_pad, d = p.shape[1], p.shape[2]
  blk = 512
  grid = n_pad // blk
  b2 = b.reshape(1, d)

  def body(p_ref, w_ref, b_ref, o_ref):
    s = p_ref[0] + p_ref[1]
    y = jnp.dot(s, w_ref[...], preferred_element_type=jnp.float32)
    y = y + b_ref[...]
    if leaky:
      y = jnp.where(y >= 0, y, 0.1 * y)
    o_ref[0] = y
    o_ref[1] = y

  return pl.pallas_call(
      body,
      grid=(grid,),
      in_specs=[
          pl.BlockSpec((NUM_CORES, blk, d), lambda i: (0, i, 0)),
          pl.BlockSpec((d, d), lambda i: (0, 0)),
          pl.BlockSpec((1, d), lambda i: (0, 0)),
      ],
      out_specs=pl.BlockSpec((NUM_CORES, blk, d), lambda i: (0, i, 0)),
      out_shape=jax.ShapeDtypeStruct((NUM_CORES, n_pad, d), jnp.float32),
  )(p, w, b2)


def kernel(in_feat, edge_index, W1, b1, W2, b2, W3, b3):
  n, d = in_feat.shape
  e = edge_index.shape[1]

  cpt = -(-e // (NW * CHUNK))
  cpt += (-cpt) % KB
  e_pad = NW * CHUNK * cpt
  n_pad = NUM_SUBCORES * KB * CHUNK
  assert n_pad > n

  src = jnp.concatenate(
      [edge_index[0], jnp.zeros((e_pad - e,), jnp.int32)]).reshape(
          NW, cpt, CHUNK)
  core_off = jnp.where(jnp.arange(NW) >= NUM_SUBCORES, n_pad, 0).astype(
      jnp.int32)
  src = src + core_off[:, None, None]
  dst = jnp.concatenate(
      [edge_index[1], jnp.full((e_pad - e,), n, jnp.int32)]).reshape(
          NW, cpt, CHUNK)
  h0 = jnp.pad(in_feat, ((0, n_pad - n), (0, 0)))
  h = jnp.concatenate([h0, h0], axis=0)

  p = _segsum(h, src, dst, n_pad, cpt)
  h = _linear(p, W1, b1, True).reshape(2 * n_pad, d)
  p = _segsum(h, src, dst, n_pad, cpt)
  h = _linear(p, W2, b2, True).reshape(2 * n_pad, d)
  p = _segsum(h, src, dst, n_pad, cpt)
  h = _linear(p, W3, b3, False)
  return h[0, :n]


# speedup vs baseline: 1.0800x; 1.0800x over previous
import functools

import jax
import jax.numpy as jnp
from jax import lax
from jax.experimental import pallas as pl
from jax.experimental.pallas import tpu as pltpu
from jax.experimental.pallas import tpu_sc as plsc

NUM_CORES = 2
NUM_SUBCORES = 16
NW = NUM_CORES * NUM_SUBCORES
CHUNK = 128
KB = 5
P = 40


@functools.partial(jax.jit, static_argnums=(3, 4))
def _segsum(h, src, dst, n_pad, cpt):
  d = h.shape[1]
  rows_per_tile = n_pad // NUM_SUBCORES
  assert rows_per_tile == KB * CHUNK

  mesh = plsc.VectorSubcoreMesh(
      core_axis_name="c", subcore_axis_name="s",
      num_cores=NUM_CORES, num_subcores=NUM_SUBCORES)

  @functools.partial(
      pl.kernel,
      out_type=jax.ShapeDtypeStruct((NUM_CORES, n_pad, d), jnp.float32),
      mesh=mesh,
      scratch_types=[
          pltpu.VMEM((P, CHUNK), jnp.int32),
          pltpu.VMEM((P, CHUNK), jnp.int32),
          pltpu.VMEM((CHUNK, d), jnp.float32),
          pltpu.VMEM((CHUNK, d), jnp.float32),
          pltpu.VMEM_SHARED((n_pad, d), jnp.float32),
          pltpu.SemaphoreType.DMA,
      ],
  )
  def seg(h_hbm, src_hbm, dst_hbm, out_hbm, src_v, dst_v, msgs, msgs1, agg,
          sem):
    cid = lax.axis_index("c")
    sid = lax.axis_index("s")
    wid = cid * NUM_SUBCORES + sid

    zero = jnp.zeros((16,), jnp.float32)

    def zbody(r, carry):
      for k in range(d // 16):
        msgs[r, pl.ds(k * 16, 16)] = zero
      return carry

    lax.fori_loop(0, CHUNK, zbody, 0)
    base = sid * rows_per_tile

    def zcopy(i, carry):
      pltpu.sync_copy(msgs, agg.at[pl.ds(base + i * CHUNK, CHUNK)])
      return carry

    lax.fori_loop(0, KB, zcopy, 0)
    plsc.subcore_barrier()

    np2 = P // 2

    def group(g, carry):
      pltpu.sync_copy(src_hbm.at[wid, pl.ds(g * P, P)], src_v)
      pltpu.sync_copy(dst_hbm.at[wid, pl.ds(g * P, P)], dst_v)
      pltpu.async_copy(h_hbm.at[src_v.at[0]], msgs, sem)

      def body(j2, c):
        l0 = 2 * j2
        pltpu.make_async_copy(h_hbm.at[src_v.at[l0]], msgs, sem).wait()
        pltpu.async_copy(h_hbm.at[src_v.at[l0 + 1]], msgs1, sem)
        pltpu.sync_copy(msgs, agg.at[dst_v.at[l0]], add=True)

        pltpu.make_async_copy(h_hbm.at[src_v.at[l0 + 1]], msgs1, sem).wait()

        @pl.when(j2 + 1 < np2)
        def _():
          pltpu.async_copy(h_hbm.at[src_v.at[l0 + 2]], msgs, sem)

        pltpu.sync_copy(msgs1, agg.at[dst_v.at[l0 + 1]], add=True)
        return c

      lax.fori_loop(0, np2, body, 0)
      return carry

    lax.fori_loop(0, cpt // P, group, 0)
    plsc.subcore_barrier()

    def ocopy(i, carry):
      sl = pl.ds(base + i * CHUNK, CHUNK)
      pltpu.sync_copy(agg.at[sl], out_hbm.at[cid, sl])
      return carry

    lax.fori_loop(0, KB, ocopy, 0)

  return seg(h, src, dst)


@functools.partial(jax.jit, static_argnums=(3,))
def _linear(p, w, b, leaky):
  n_pad, d = p.shape[1], p.shape[2]
  blk = 512
  grid = n_pad // blk
  b2 = b.reshape(1, d)

  def body(p_ref, w_ref, b_ref, o_ref):
    s = p_ref[0] + p_ref[1]
    y = jnp.dot(s, w_ref[...], preferred_element_type=jnp.float32)
    y = y + b_ref[...]
    if leaky:
      y = jnp.where(y >= 0, y, 0.1 * y)
    o_ref[0] = y
    o_ref[1] = y

  return pl.pallas_call(
      body,
      grid=(grid,),
      in_specs=[
          pl.BlockSpec((NUM_CORES, blk, d), lambda i: (0, i, 0)),
          pl.BlockSpec((d, d), lambda i: (0, 0)),
          pl.BlockSpec((1, d), lambda i: (0, 0)),
      ],
      out_specs=pl.BlockSpec((NUM_CORES, blk, d), lambda i: (0, i, 0)),
      out_shape=jax.ShapeDtypeStruct((NUM_CORES, n_pad, d), jnp.float32),
  )(p, w, b2)


def kernel(in_feat, edge_index, W1, b1, W2, b2, W3, b3):
  n, d = in_feat.shape
  e = edge_index.shape[1]

  cpt = -(-e // (NW * CHUNK))
  cpt += (-cpt) % P
  e_pad = NW * CHUNK * cpt
  n_pad = NUM_SUBCORES * KB * CHUNK
  assert n_pad > n

  src = jnp.concatenate(
      [edge_index[0], jnp.zeros((e_pad - e,), jnp.int32)]).reshape(
          NW, cpt, CHUNK)
  core_off = jnp.where(jnp.arange(NW) >= NUM_SUBCORES, n_pad, 0).astype(
      jnp.int32)
  src = src + core_off[:, None, None]
  dst = jnp.concatenate(
      [edge_index[1], jnp.full((e_pad - e,), n, jnp.int32)]).reshape(
          NW, cpt, CHUNK)
  h0 = jnp.pad(in_feat, ((0, n_pad - n), (0, 0)))
  h = jnp.concatenate([h0, h0], axis=0)

  p = _segsum(h, src, dst, n_pad, cpt)
  h = _linear(p, W1, b1, True).reshape(2 * n_pad, d)
  p = _segsum(h, src, dst, n_pad, cpt)
  h = _linear(p, W2, b2, True).reshape(2 * n_pad, d)
  p = _segsum(h, src, dst, n_pad, cpt)
  h = _linear(p, W3, b3, False)
  return h[0, :n]
